# baseline (speedup 1.0000x reference)
import functools

import jax
import jax.numpy as jnp
from jax import lax
from jax.experimental import pallas as pl
from jax.experimental.pallas import tpu as pltpu
from jax.experimental.pallas import tpu_sc as plsc

_CHUNK = 128
_LANE = 16


def _sc_title_pool(tok_flat, title_z, B):
  E = title_z.shape[1]
  TL = tok_flat.shape[0] // B

  info = plsc.get_sparse_core_info()
  nw = info.num_cores * info.num_subcores
  b_per_w = B // nw
  n_chunks = b_per_w // _CHUNK
  mesh = plsc.VectorSubcoreMesh(core_axis_name="c", subcore_axis_name="s")

  @functools.partial(
      pl.kernel,
      out_type=jax.ShapeDtypeStruct((B, E), jnp.float32),
      mesh=mesh,
      compiler_params=pltpu.CompilerParams(use_tc_tiling_on_sc=False),
      scratch_types=[
          pltpu.VMEM((TL, b_per_w), jnp.int32),
          pltpu.VMEM((b_per_w, E), jnp.float32),
          pltpu.SemaphoreType.DMA,
          pltpu.SemaphoreType.DMA,
      ],
  )
  def sck(tok_hbm, ttab_hbm, out_hbm, tok_v, tacc_v, sem_a, sem_b):
    wid = lax.axis_index("s") * info.num_cores + lax.axis_index("c")
    base = wid * b_per_w
    stage = [
        pltpu.async_copy(tok_hbm.at[pl.ds(t * B + base, b_per_w)],
                         tok_v.at[t], sem_a)
        for t in range(TL)
    ]
    for cp in stage:
      cp.wait()

    def chunk_body(ci, carry):
      sl = pl.ds(ci * _CHUNK, _CHUNK)
      cp_t0 = pltpu.async_copy(ttab_hbm.at[tok_v.at[0, sl]],
                               tacc_v.at[sl], sem_a)
      cp_t0.wait()
      adds = [
          pltpu.async_copy(ttab_hbm.at[tok_v.at[t, sl]],
                           tacc_v.at[sl], sem_b, add=True)
          for t in range(1, TL)
      ]
      for cp in adds:
        cp.wait()
      return carry

    lax.fori_loop(0, n_chunks, chunk_body, 0)
    pltpu.sync_copy(tacc_v, out_hbm.at[pl.ds(base, b_per_w), :])

  return sck(tok_flat, title_z)


def _sc_movie_gather(movie_id, mtab_T):
  B = movie_id.shape[0]
  E, V = mtab_T.shape
  full = (V // _CHUNK) * _CHUNK
  ns = _LANE

  info = plsc.get_sparse_core_info()
  nw = info.num_cores * info.num_subcores
  b_per_w = B // nw
  n_sub = b_per_w // ns
  half_rows = b_per_w // 2
  mesh = plsc.VectorSubcoreMesh(core_axis_name="c", subcore_axis_name="s")

  @functools.partial(
      pl.kernel,
      out_type=jax.ShapeDtypeStruct((B, 4 * E), jnp.float32),
      mesh=mesh,
      compiler_params=pltpu.CompilerParams(use_tc_tiling_on_sc=True,
                                           needs_layout_passes=False),
      scratch_types=[
          pltpu.VMEM((b_per_w,), jnp.int32),
          pltpu.VMEM((ns * E, _CHUNK), jnp.float32),
          pltpu.VMEM((b_per_w // 2, 4 * E), jnp.float32),
          [pltpu.SemaphoreType.DMA] * ns,
      ],
  )
  def sck(mid_hbm, mtab_hbm, out_hbm, mid_v, grp_v, mrows_v, sems):
    wid = lax.axis_index("s") * info.num_cores + lax.axis_index("c")
    base = wid * b_per_w
    pltpu.sync_copy(mid_hbm.at[pl.ds(base, b_per_w)], mid_v)
    rows = lax.iota(jnp.int32, _LANE)

    def lanes_of(s):
      idv = jnp.minimum(mid_v[pl.ds(s * ns, _LANE)], full - 1)
      return (idv >> 7) * _CHUNK, idv & (_CHUNK - 1)

    def fire(j, gs):
      return pltpu.async_copy(
          mtab_hbm.at[:, pl.ds(pl.multiple_of(gs[j], _CHUNK), _CHUNK)],
          grp_v.at[pl.ds(j * E, E), :], sems[j])

    def drain(j):
      pltpu.make_async_copy(mtab_hbm.at[:, pl.ds(0, _CHUNK)],
                            grp_v.at[pl.ds(j * E, E), :], sems[j]).wait()

    def extract(j, lane, r):
      col = jnp.full((_LANE,), lane[j], jnp.int32)
      for k in range(E // _LANE):
        seg = plsc.load_gather(grp_v, [rows + (j * E + k * _LANE), col])
        mrows_v[r & (half_rows - 1), pl.ds(k * _LANE, _LANE)] = seg

    gs0, lane0 = lanes_of(0)
    for j in range(ns):
      fire(j, gs0)

    def sub_body(s, lane_prev):
      @pl.when(s * ns - ns == half_rows)
      def _():
        pltpu.sync_copy(mrows_v, out_hbm.at[pl.ds(base, half_rows), :])

      gs, lane = lanes_of(s)
      for j in range(ns):
        drain(j)
        extract(j, lane_prev, s * ns - ns + j)
        fire(j, gs)
      return lane

    lane_last = lax.fori_loop(1, n_sub, sub_body, lane0)
    for j in range(ns):
      drain(j)
      extract(j, lane_last, b_per_w - ns + j)
    pltpu.sync_copy(mrows_v, out_hbm.at[pl.ds(base + half_rows, half_rows), :])

  return sck(movie_id, mtab_T)


def _mlp_body(xbase, tsum_ref, emov_ref, mid_ref, ttl_ref, gen_ref, gtab_ref,
              xtab_ref, w1_ref, b1_ref, w2_ref, b2_ref, w3_ref, b3_ref,
              out_ref):
  f32 = jnp.float32
  e = gtab_ref.shape[1]
  tmask = (ttl_ref[...] != 0).astype(f32)
  tcnt = jnp.maximum(jnp.sum(tmask, axis=1, keepdims=True), 1.0)
  e_title = tsum_ref[:, 0:e] / tcnt

  mid = mid_ref[...]
  nx = xtab_ref.shape[0]
  bb = mid.shape[0]
  xiota = lax.broadcasted_iota(jnp.int32, (bb, nx), 1)
  xoh = ((mid - xbase) == xiota).astype(f32)
  xrows = jnp.dot(xoh, xtab_ref[...], preferred_element_type=f32)
  tail = (mid >= xbase).astype(f32)
  e_movie = emov_ref[:, 0:e] * (1.0 - tail) + xrows * tail

  gen = gen_ref[...]
  ng = gtab_ref.shape[0]
  iota = lax.broadcasted_iota(jnp.int32, (bb, ng), 1)
  counts = jnp.zeros((bb, ng), f32)
  gcnt = jnp.zeros((bb, 1), f32)
  for t in range(gen.shape[1]):
    col = gen[:, t:t + 1]
    counts = counts + (col == iota).astype(f32)
    gcnt = gcnt + (col != 0).astype(f32)
  gsum = jnp.dot(counts, gtab_ref[...], preferred_element_type=f32)
  e_genre = gsum / jnp.maximum(gcnt, 1.0)

  w1 = w1_ref[...]
  h = (jnp.dot(e_movie, w1[0:e], preferred_element_type=f32)
       + jnp.dot(e_title, w1[e:2 * e], preferred_element_type=f32)
       + jnp.dot(e_genre, w1[2 * e:3 * e], preferred_element_type=f32)
       + b1_ref[...])
  h = jnp.maximum(h, 0.0)
  h = jnp.maximum(jnp.dot(h, w2_ref[...], preferred_element_type=f32)
                  + b2_ref[...], 0.0)
  out_ref[...] = (jnp.dot(h, w3_ref[...], preferred_element_type=f32)
                  + b3_ref[...])


def _tc_mlp(t_sum, e_mov, movie_id2, titles, genres, genre_table_z,
            extra_tab, xbase, W1, b1, W2, b2, W3, b3, block_b=512):
  B, ET = t_sum.shape
  EP = e_mov.shape[1]
  E = genre_table_z.shape[1]
  TL = titles.shape[1]
  GL = genres.shape[1]
  NG = genre_table_z.shape[0]
  NX = extra_tab.shape[0]
  H1 = W1.shape[1]
  H2 = W2.shape[1]
  DO = W3.shape[1]
  grid = (B // block_b,)
  whole = lambda shape: pl.BlockSpec(shape, lambda i: (0, 0))
  blk = lambda cols: pl.BlockSpec((block_b, cols), lambda i: (i, 0))
  return pl.pallas_call(
      functools.partial(_mlp_body, int(xbase)),
      grid=grid,
      in_specs=[
          blk(ET), blk(EP), blk(1), blk(TL), blk(GL), whole((NG, E)),
          whole((NX, E)),
          whole((3 * E, H1)), whole((1, H1)),
          whole((H1, H2)), whole((1, H2)),
          whole((H2, DO)), whole((1, DO)),
      ],
      out_specs=blk(DO),
      out_shape=jax.ShapeDtypeStruct((B, DO), jnp.float32),
  )(t_sum, e_mov, movie_id2, titles, genres, genre_table_z, extra_tab,
    W1, b1.reshape(1, -1), W2, b2.reshape(1, -1), W3, b3.reshape(1, -1))


def kernel(movie_id, movie_title_vector, movie_genres, movie_table,
           title_table, genre_table, W1, b1, W2, b2, W3, b3):
  B = movie_id.shape[0]
  V, E = movie_table.shape
  title_z = title_table.at[0].set(0.0)
  genre_z = genre_table.at[0].set(0.0)
  tok_flat = movie_title_vector.T.reshape(-1)
  mid32 = movie_id.astype(jnp.int32)
  xbase = V // 128 * 128
  nx = V - xbase
  extra_tab = jnp.pad(movie_table[xbase:], ((0, (-nx) % 8), (0, 0)))

  t_sum = _sc_title_pool(tok_flat, title_z, B)
  e_mov = _sc_movie_gather(mid32, movie_table.T)
  return _tc_mlp(t_sum, e_mov, mid32.reshape(B, 1), movie_title_vector,
                 movie_genres, genre_z, extra_tab, xbase,
                 W1, b1, W2, b2, W3, b3)


# speedup vs baseline: 1.0100x; 1.0100x over previous
import functools

import jax
import jax.numpy as jnp
from jax import lax
from jax.experimental import pallas as pl
from jax.experimental.pallas import tpu as pltpu
from jax.experimental.pallas import tpu_sc as plsc

_CHUNK = 128
_LANE = 16


def _sc_title_pool(tok_flat, title_z, B):
  E = title_z.shape[1]
  TL = tok_flat.shape[0] // B

  info = plsc.get_sparse_core_info()
  nw = info.num_cores * info.num_subcores
  b_per_w = B // nw
  n_chunks = b_per_w // _CHUNK
  mesh = plsc.VectorSubcoreMesh(core_axis_name="c", subcore_axis_name="s")

  @functools.partial(
      pl.kernel,
      out_type=jax.ShapeDtypeStruct((B, E), jnp.float32),
      mesh=mesh,
      compiler_params=pltpu.CompilerParams(use_tc_tiling_on_sc=False),
      scratch_types=[
          pltpu.VMEM((TL, b_per_w), jnp.int32),
          pltpu.VMEM((b_per_w, E), jnp.float32),
          pltpu.SemaphoreType.DMA,
          pltpu.SemaphoreType.DMA,
      ],
  )
  def sck(tok_hbm, ttab_hbm, out_hbm, tok_v, tacc_v, sem_a, sem_b):
    wid = lax.axis_index("s") * info.num_cores + lax.axis_index("c")
    base = wid * b_per_w
    stage = [
        pltpu.async_copy(tok_hbm.at[pl.ds(t * B + base, b_per_w)],
                         tok_v.at[t], sem_a)
        for t in range(TL)
    ]
    for cp in stage:
      cp.wait()

    def chunk_body(ci, carry):
      sl = pl.ds(ci * _CHUNK, _CHUNK)
      cp_t0 = pltpu.async_copy(ttab_hbm.at[tok_v.at[0, sl]],
                               tacc_v.at[sl], sem_a)
      cp_t0.wait()
      adds = [
          pltpu.async_copy(ttab_hbm.at[tok_v.at[t, sl]],
                           tacc_v.at[sl], sem_b, add=True)
          for t in range(1, TL)
      ]
      for cp in adds:
        cp.wait()
      return carry

    lax.fori_loop(0, n_chunks, chunk_body, 0)
    pltpu.sync_copy(tacc_v, out_hbm.at[pl.ds(base, b_per_w), :])

  return sck(tok_flat, title_z)


def _sc_movie_gather(movie_id, mtab_T):
  B = movie_id.shape[0]
  E, V = mtab_T.shape
  full = (V // _CHUNK) * _CHUNK
  ns = 8

  info = plsc.get_sparse_core_info()
  nw = info.num_cores * info.num_subcores
  b_per_w = B // nw
  n_sub = b_per_w // ns
  mesh = plsc.VectorSubcoreMesh(core_axis_name="c", subcore_axis_name="s")

  @functools.partial(
      pl.kernel,
      out_type=jax.ShapeDtypeStruct((B, 4 * E), jnp.float32),
      mesh=mesh,
      compiler_params=pltpu.CompilerParams(use_tc_tiling_on_sc=True,
                                           needs_layout_passes=False),
      scratch_types=[
          pltpu.VMEM((b_per_w + _LANE,), jnp.int32),
          pltpu.VMEM((ns * E, _CHUNK), jnp.float32),
          pltpu.VMEM((b_per_w, 4 * E), jnp.float32),
          [pltpu.SemaphoreType.DMA] * ns,
      ],
  )
  def sck(mid_hbm, mtab_hbm, out_hbm, mid_v, grp_v, mrows_v, sems):
    wid = lax.axis_index("s") * info.num_cores + lax.axis_index("c")
    base = wid * b_per_w
    pltpu.sync_copy(mid_hbm.at[pl.ds(base, b_per_w)],
                    mid_v.at[pl.ds(0, b_per_w)])
    rows = lax.iota(jnp.int32, _LANE)

    def lanes_of(s):
      idv = jnp.minimum(mid_v[pl.ds(s * ns, _LANE)], full - 1)
      return (idv >> 7) * _CHUNK, idv & (_CHUNK - 1)

    def fire(j, gs):
      return pltpu.async_copy(
          mtab_hbm.at[:, pl.ds(pl.multiple_of(gs[j], _CHUNK), _CHUNK)],
          grp_v.at[pl.ds(j * E, E), :], sems[j])

    def drain(j):
      pltpu.make_async_copy(mtab_hbm.at[:, pl.ds(0, _CHUNK)],
                            grp_v.at[pl.ds(j * E, E), :], sems[j]).wait()

    def extract(j, lane, r):
      col = jnp.full((_LANE,), lane[j], jnp.int32)
      for k in range(E // _LANE):
        seg = plsc.load_gather(grp_v, [rows + (j * E + k * _LANE), col])
        mrows_v[r, pl.ds(k * _LANE, _LANE)] = seg

    gs0, lane0 = lanes_of(0)
    for j in range(ns):
      fire(j, gs0)

    def sub_body(s, lane_prev):
      gs, lane = lanes_of(s)
      for j in range(ns):
        drain(j)
        extract(j, lane_prev, s * ns - ns + j)
        fire(j, gs)
      return lane

    lane_last = lax.fori_loop(1, n_sub, sub_body, lane0)
    for j in range(ns):
      drain(j)
      extract(j, lane_last, b_per_w - ns + j)
    pltpu.sync_copy(mrows_v, out_hbm.at[pl.ds(base, b_per_w), :])

  return sck(movie_id, mtab_T)


def _mlp_body(xbase, tsum_ref, emov_ref, mid_ref, ttl_ref, gen_ref, gtab_ref,
              xtab_ref, w1_ref, b1_ref, w2_ref, b2_ref, w3_ref, b3_ref,
              out_ref):
  f32 = jnp.float32
  e = gtab_ref.shape[1]
  tmask = (ttl_ref[...] != 0).astype(f32)
  tcnt = jnp.maximum(jnp.sum(tmask, axis=1, keepdims=True), 1.0)
  e_title = tsum_ref[:, 0:e] / tcnt

  mid = mid_ref[...]
  nx = xtab_ref.shape[0]
  bb = mid.shape[0]
  xiota = lax.broadcasted_iota(jnp.int32, (bb, nx), 1)
  xoh = ((mid - xbase) == xiota).astype(f32)
  xrows = jnp.dot(xoh, xtab_ref[...], preferred_element_type=f32)
  tail = (mid >= xbase).astype(f32)
  e_movie = emov_ref[:, 0:e] * (1.0 - tail) + xrows * tail

  gen = gen_ref[...]
  ng = gtab_ref.shape[0]
  iota = lax.broadcasted_iota(jnp.int32, (bb, ng), 1)
  counts = jnp.zeros((bb, ng), f32)
  gcnt = jnp.zeros((bb, 1), f32)
  for t in range(gen.shape[1]):
    col = gen[:, t:t + 1]
    counts = counts + (col == iota).astype(f32)
    gcnt = gcnt + (col != 0).astype(f32)
  gsum = jnp.dot(counts, gtab_ref[...], preferred_element_type=f32)
  e_genre = gsum / jnp.maximum(gcnt, 1.0)

  w1 = w1_ref[...]
  h = (jnp.dot(e_movie, w1[0:e], preferred_element_type=f32)
       + jnp.dot(e_title, w1[e:2 * e], preferred_element_type=f32)
       + jnp.dot(e_genre, w1[2 * e:3 * e], preferred_element_type=f32)
       + b1_ref[...])
  h = jnp.maximum(h, 0.0)
  h = jnp.maximum(jnp.dot(h, w2_ref[...], preferred_element_type=f32)
                  + b2_ref[...], 0.0)
  out_ref[...] = (jnp.dot(h, w3_ref[...], preferred_element_type=f32)
                  + b3_ref[...])


def _tc_mlp(t_sum, e_mov, movie_id2, titles, genres, genre_table_z,
            extra_tab, xbase, W1, b1, W2, b2, W3, b3, block_b=512):
  B, ET = t_sum.shape
  EP = e_mov.shape[1]
  E = genre_table_z.shape[1]
  TL = titles.shape[1]
  GL = genres.shape[1]
  NG = genre_table_z.shape[0]
  NX = extra_tab.shape[0]
  H1 = W1.shape[1]
  H2 = W2.shape[1]
  DO = W3.shape[1]
  grid = (B // block_b,)
  whole = lambda shape: pl.BlockSpec(shape, lambda i: (0, 0))
  blk = lambda cols: pl.BlockSpec((block_b, cols), lambda i: (i, 0))
  return pl.pallas_call(
      functools.partial(_mlp_body, int(xbase)),
      grid=grid,
      in_specs=[
          blk(ET), blk(EP), blk(1), blk(TL), blk(GL), whole((NG, E)),
          whole((NX, E)),
          whole((3 * E, H1)), whole((1, H1)),
          whole((H1, H2)), whole((1, H2)),
          whole((H2, DO)), whole((1, DO)),
      ],
      out_specs=blk(DO),
      out_shape=jax.ShapeDtypeStruct((B, DO), jnp.float32),
  )(t_sum, e_mov, movie_id2, titles, genres, genre_table_z, extra_tab,
    W1, b1.reshape(1, -1), W2, b2.reshape(1, -1), W3, b3.reshape(1, -1))


def kernel(movie_id, movie_title_vector, movie_genres, movie_table,
           title_table, genre_table, W1, b1, W2, b2, W3, b3):
  B = movie_id.shape[0]
  V, E = movie_table.shape
  title_z = title_table.at[0].set(0.0)
  genre_z = genre_table.at[0].set(0.0)
  tok_flat = movie_title_vector.T.reshape(-1)
  mid32 = movie_id.astype(jnp.int32)
  xbase = V // 128 * 128
  nx = V - xbase
  extra_tab = jnp.pad(movie_table[xbase:], ((0, (-nx) % 8), (0, 0)))

  t_sum = _sc_title_pool(tok_flat, title_z, B)
  e_mov = _sc_movie_gather(mid32, movie_table.T)
  return _tc_mlp(t_sum, e_mov, mid32.reshape(B, 1), movie_title_vector,
                 movie_genres, genre_z, extra_tab, xbase,
                 W1, b1, W2, b2, W3, b3)


# speedup vs baseline: 1.0610x; 1.0505x over previous
import functools

import jax
import jax.numpy as jnp
from jax import lax
from jax.experimental import pallas as pl
from jax.experimental.pallas import tpu as pltpu
from jax.experimental.pallas import tpu_sc as plsc

_CHUNK = 128
_LANE = 16


def _sc_title_pool(tok_flat, title_z, B):
  E = title_z.shape[1]
  TL = tok_flat.shape[0] // B

  info = plsc.get_sparse_core_info()
  nw = info.num_cores * info.num_subcores
  b_per_w = B // nw
  n_chunks = b_per_w // _CHUNK
  mesh = plsc.VectorSubcoreMesh(core_axis_name="c", subcore_axis_name="s")

  @functools.partial(
      pl.kernel,
      out_type=jax.ShapeDtypeStruct((B, E), jnp.float32),
      mesh=mesh,
      compiler_params=pltpu.CompilerParams(use_tc_tiling_on_sc=False),
      scratch_types=[
          pltpu.VMEM((TL, b_per_w), jnp.int32),
          pltpu.VMEM((b_per_w, E), jnp.float32),
          pltpu.SemaphoreType.DMA,
          pltpu.SemaphoreType.DMA,
      ],
  )
  def sck(tok_hbm, ttab_hbm, out_hbm, tok_v, tacc_v, sem_a, sem_b):
    wid = lax.axis_index("s") * info.num_cores + lax.axis_index("c")
    base = wid * b_per_w
    stage = [
        pltpu.async_copy(tok_hbm.at[pl.ds(t * B + base, b_per_w)],
                         tok_v.at[t], sem_a)
        for t in range(TL)
    ]
    for cp in stage:
      cp.wait()

    def chunk_body(ci, carry):
      sl = pl.ds(ci * _CHUNK, _CHUNK)
      cp_t0 = pltpu.async_copy(ttab_hbm.at[tok_v.at[0, sl]],
                               tacc_v.at[sl], sem_a)
      cp_t0.wait()
      adds = [
          pltpu.async_copy(ttab_hbm.at[tok_v.at[t, sl]],
                           tacc_v.at[sl], sem_b, add=True)
          for t in range(1, TL)
      ]
      for cp in adds:
        cp.wait()
      return carry

    lax.fori_loop(0, n_chunks, chunk_body, 0)
    pltpu.sync_copy(tacc_v, out_hbm.at[pl.ds(base, b_per_w), :])

  return sck(tok_flat, title_z)


def _sc_movie_gather(movie_id, mtab_T):
  B = movie_id.shape[0]
  E, V = mtab_T.shape
  full = (V // _CHUNK) * _CHUNK
  ns = 8

  info = plsc.get_sparse_core_info()
  nw = info.num_cores * info.num_subcores
  b_per_w = B // nw
  n_sub = b_per_w // ns
  mesh = plsc.VectorSubcoreMesh(core_axis_name="c", subcore_axis_name="s")

  @functools.partial(
      pl.kernel,
      out_type=jax.ShapeDtypeStruct((B, 4 * E), jnp.float32),
      mesh=mesh,
      compiler_params=pltpu.CompilerParams(use_tc_tiling_on_sc=True,
                                           needs_layout_passes=False),
      scratch_types=[
          pltpu.VMEM((b_per_w + _LANE,), jnp.int32),
          pltpu.VMEM((ns * E, _CHUNK), jnp.float32),
          pltpu.VMEM((b_per_w, 4 * E), jnp.float32),
          [pltpu.SemaphoreType.DMA] * ns,
      ],
  )
  def sck(mid_hbm, mtab_hbm, out_hbm, mid_v, grp_v, mrows_v, sems):
    wid = lax.axis_index("s") * info.num_cores + lax.axis_index("c")
    base = wid * b_per_w
    pltpu.sync_copy(mid_hbm.at[pl.ds(base, b_per_w)],
                    mid_v.at[pl.ds(0, b_per_w)])
    rows = lax.iota(jnp.int32, _LANE)

    def lanes_of(s):
      idv = jnp.minimum(mid_v[pl.ds(s * ns, _LANE)], full - 1)
      return (idv >> 7) * _CHUNK, idv & (_CHUNK - 1)

    def fire(j, gs):
      return pltpu.async_copy(
          mtab_hbm.at[:, pl.ds(pl.multiple_of(gs[j], _CHUNK), _CHUNK)],
          grp_v.at[pl.ds(j * E, E), :], sems[j])

    def drain(j):
      pltpu.make_async_copy(mtab_hbm.at[:, pl.ds(0, _CHUNK)],
                            grp_v.at[pl.ds(j * E, E), :], sems[j]).wait()

    def extract(j, lane, r):
      col = jnp.full((_LANE,), lane[j], jnp.int32)
      for k in range(E // _LANE):
        seg = plsc.load_gather(grp_v, [rows + (j * E + k * _LANE), col])
        mrows_v[r, pl.ds(k * _LANE, _LANE)] = seg

    gs0, lane0 = lanes_of(0)
    for j in range(ns):
      fire(j, gs0)

    def sub_body(s, lane_prev):
      gs, lane = lanes_of(s)
      for j in range(ns):
        drain(j)
        extract(j, lane_prev, s * ns - ns + j)
        fire(j, gs)
      return lane

    lane_last = lax.fori_loop(1, n_sub, sub_body, lane0)
    for j in range(ns):
      drain(j)
      extract(j, lane_last, b_per_w - ns + j)
    pltpu.sync_copy(mrows_v, out_hbm.at[pl.ds(base, b_per_w), :])

  return sck(movie_id, mtab_T)


def _mlp_body(xbase, tsum_ref, emov_ref, mid_ref, ttl_ref, gen_ref, gtab_ref,
              xtab_ref, w1_ref, b1_ref, w2_ref, b2_ref, w3_ref, b3_ref,
              out_ref):
  f32 = jnp.float32
  e = gtab_ref.shape[1]
  tmask = (ttl_ref[...] != 0).astype(f32)
  tcnt = jnp.maximum(jnp.sum(tmask, axis=1, keepdims=True), 1.0)
  e_title = tsum_ref[:, 0:e] / tcnt

  mid = mid_ref[...]
  nx = xtab_ref.shape[0]
  bb = mid.shape[0]
  xiota = lax.broadcasted_iota(jnp.int32, (bb, nx), 1)
  xoh = ((mid - xbase) == xiota).astype(f32)
  xrows = jnp.dot(xoh, xtab_ref[...], preferred_element_type=f32)
  tail = (mid >= xbase).astype(f32)
  e_movie = emov_ref[:, 0:e] * (1.0 - tail) + xrows * tail

  gen = gen_ref[...]
  ng = gtab_ref.shape[0]
  iota = lax.broadcasted_iota(jnp.int32, (bb, ng), 1)
  counts = jnp.zeros((bb, ng), f32)
  gcnt = jnp.zeros((bb, 1), f32)
  for t in range(gen.shape[1]):
    col = gen[:, t:t + 1]
    counts = counts + (col == iota).astype(f32)
    gcnt = gcnt + (col != 0).astype(f32)
  gsum = jnp.dot(counts, gtab_ref[...], preferred_element_type=f32)
  e_genre = gsum / jnp.maximum(gcnt, 1.0)

  w1 = w1_ref[...]
  h = (jnp.dot(e_movie, w1[0:e], preferred_element_type=f32)
       + jnp.dot(e_title, w1[e:2 * e], preferred_element_type=f32)
       + jnp.dot(e_genre, w1[2 * e:3 * e], preferred_element_type=f32)
       + b1_ref[...])
  h = jnp.maximum(h, 0.0)
  h = jnp.maximum(jnp.dot(h, w2_ref[...], preferred_element_type=f32)
                  + b2_ref[...], 0.0)
  out_ref[...] = (jnp.dot(h, w3_ref[...], preferred_element_type=f32)
                  + b3_ref[...])


def _tc_mlp(t_sum, e_mov, movie_id2, titles, genres, genre_table_z,
            extra_tab, xbase, W1, b1, W2, b2, W3, b3, block_b=512):
  B, ET = t_sum.shape
  EP = e_mov.shape[1]
  E = genre_table_z.shape[1]
  TL = titles.shape[1]
  GL = genres.shape[1]
  NG = genre_table_z.shape[0]
  NX = extra_tab.shape[0]
  H1 = W1.shape[1]
  H2 = W2.shape[1]
  DO = W3.shape[1]
  grid = (B // block_b,)
  whole = lambda shape: pl.BlockSpec(shape, lambda i: (0, 0))
  blk = lambda cols: pl.BlockSpec((block_b, cols), lambda i: (i, 0))
  return pl.pallas_call(
      functools.partial(_mlp_body, int(xbase)),
      grid=grid,
      in_specs=[
          blk(ET), blk(EP), blk(1), blk(TL), blk(GL), whole((NG, E)),
          whole((NX, E)),
          whole((3 * E, H1)), whole((1, H1)),
          whole((H1, H2)), whole((1, H2)),
          whole((H2, DO)), whole((1, DO)),
      ],
      out_specs=blk(DO),
      out_shape=jax.ShapeDtypeStruct((B, DO), jnp.float32),
  )(t_sum, e_mov, movie_id2, titles, genres, genre_table_z, extra_tab,
    W1, b1.reshape(1, -1), W2, b2.reshape(1, -1), W3, b3.reshape(1, -1))


def kernel(movie_id, movie_title_vector, movie_genres, movie_table,
           title_table, genre_table, W1, b1, W2, b2, W3, b3):
  B = movie_id.shape[0]
  V, E = movie_table.shape
  title_z = title_table.at[0].set(0.0)
  genre_z = genre_table.at[0].set(0.0)
  mid32 = movie_id.astype(jnp.int32)
  xbase = V // 128 * 128
  nx = V - xbase
  extra_tab = jnp.pad(movie_table[xbase:], ((0, (-nx) % 8), (0, 0)))

  H = B // 2
  mtab_T = movie_table.T
  outs = []
  for h in range(2):
    sl = slice(h * H, (h + 1) * H)
    tok_h = movie_title_vector[sl].T.reshape(-1)
    t_sum = _sc_title_pool(tok_h, title_z, H)
    e_mov = _sc_movie_gather(mid32[sl], mtab_T)
    outs.append(_tc_mlp(t_sum, e_mov, mid32[sl].reshape(H, 1),
                        movie_title_vector[sl], movie_genres[sl], genre_z,
                        extra_tab, xbase, W1, b1, W2, b2, W3, b3))
  return jnp.concatenate(outs, axis=0)
